# Initial kernel scaffold; baseline (speedup 1.0000x reference)
import jax
import jax.numpy as jnp
from jax.experimental import pallas as pl


def kernel(inputs):
    raise NotImplementedError("write your pallas kernel here")


# speedup vs baseline: 9.9288x; 9.9288x over previous
import functools

import jax
import jax.numpy as jnp
import numpy as np
from jax import lax
from jax.experimental import pallas as pl
from jax.experimental.pallas import tpu as pltpu
from jax.experimental.pallas import tpu_sc as plsc

R = 128
N = 32768
K = 65
L = 16
NV = N // L
NC = 2
NS = 16
NW = NC * NS
ROWS_PER_W = R // NW
PIVOT = np.float32(2.0)
SIGN = np.int32(-(2**31))
LOW31 = np.int32(0x7FFFFFFF)
INT_MIN = np.int32(-(2**31))


def _ckey(v):
    b = lax.bitcast_convert_type(v, jnp.int32)
    return jnp.where(b >= 0, b, b ^ LOW31)


def _sc_body(x_hbm, out_hbm, row_v, cand_v):
    wid = lax.axis_index("s") * NC + lax.axis_index("c")
    for j in range(ROWS_PER_W):
        r = wid * ROWS_PER_W + j
        pltpu.sync_copy(x_hbm.at[r], row_v)

        def comp_body(i, off):
            v = row_v[pl.ds(i * L, L)]
            m = v > PIVOT
            mi = m.astype(jnp.int32)
            pos = (plsc.cumsum(mi) - mi) + off
            plsc.store_scatter(cand_v, [pos], _ckey(v), mask=m)
            return off + plsc.all_reduce_population_count(m)

        offv = lax.fori_loop(0, NV, comp_body, jnp.zeros((L,), jnp.int32))
        cnt = jnp.max(offv)
        padpos = lax.iota(jnp.int32, L) + cnt
        plsc.store_scatter(cand_v, [padpos], jnp.full((L,), INT_MIN, jnp.int32))

        @pl.when(cnt < K)
        def _():
            def copy_body(i, _):
                cand_v[pl.ds(i * L, L)] = _ckey(row_v[pl.ds(i * L, L)])
                return 0
            lax.fori_loop(0, NV, copy_body, 0)

        cnt = jnp.where(cnt < K, N, cnt)
        nv = (cnt + (L - 1)) // L

        def bit_body(bi, p):
            bit = jnp.left_shift(jnp.int32(1), 31 - bi)
            cand_t = p | bit
            cs = cand_t ^ SIGN

            def cbody(i, acc):
                kv = cand_v[pl.ds(i * L, L)]
                return acc + jnp.where(kv >= cs,
                                       jnp.full((L,), 1, jnp.int32),
                                       jnp.full((L,), 0, jnp.int32))

            accv = lax.fori_loop(0, nv, cbody, jnp.zeros((L,), jnp.int32))
            return jnp.where(jnp.sum(accv) >= K, cand_t, p)

        p_u = lax.fori_loop(0, 32, bit_body, jnp.int32(0))
        vkey = p_u ^ SIGN

        def mbody(i, _):
            v = row_v[pl.ds(i * L, L)]
            row_v[pl.ds(i * L, L)] = jnp.where(_ckey(v) > vkey, v,
                                               jnp.float32(0.0))
            return 0

        lax.fori_loop(0, NV, mbody, 0)
        pltpu.sync_copy(row_v, out_hbm.at[r])


@jax.jit
def _ksparse_sc(x):
    mesh = plsc.VectorSubcoreMesh(core_axis_name="c", subcore_axis_name="s")
    return pl.kernel(
        _sc_body,
        out_type=jax.ShapeDtypeStruct((R, N), jnp.float32),
        mesh=mesh,
        compiler_params=pltpu.CompilerParams(needs_layout_passes=False),
        scratch_types=[
            pltpu.VMEM((N,), jnp.float32),
            pltpu.VMEM((N + L,), jnp.int32),
        ],
    )(x)


def kernel(inputs):
    return _ksparse_sc(inputs)


# speedup vs baseline: 33.1053x; 3.3343x over previous
import jax
import jax.numpy as jnp
import numpy as np
from jax import lax
from jax.experimental import pallas as pl
from jax.experimental.pallas import tpu as pltpu
from jax.experimental.pallas import tpu_sc as plsc

R = 128
N = 32768
K = 65
L = 16
NV = N // L
NC = 2
NS = 16
NW = NC * NS
ROWS_PER_W = R // NW
PIVOT = np.float32(2.5)
SIGN = np.int32(-(2**31))
LOW31 = np.int32(0x7FFFFFFF)
INT_MIN = np.int32(-(2**31))
UNROLL = 8


def _ckey(v):
    b = lax.bitcast_convert_type(v, jnp.int32)
    return jnp.where(b >= 0, b, b ^ LOW31)


def _row_threshold_key(row_v, cand_v):
    @plsc.parallel_loop(0, NV, unroll=UNROLL,
                        carry=jnp.zeros((L,), jnp.int32))
    def offv(i, off):
        v = row_v[pl.ds(i * L, L)]
        m = v > PIVOT
        mi = m.astype(jnp.int32)
        pos = (plsc.cumsum(mi) - mi) + off
        plsc.store_scatter(cand_v, [pos], _ckey(v), mask=m)
        return off + plsc.all_reduce_population_count(m)

    cnt = jnp.max(offv)
    padpos = lax.iota(jnp.int32, L) + cnt
    plsc.store_scatter(cand_v, [padpos], jnp.full((L,), INT_MIN, jnp.int32))

    @pl.when(cnt < K)
    def _():
        @plsc.parallel_loop(0, NV, unroll=UNROLL)
        def _copy(i):
            cand_v[pl.ds(i * L, L)] = _ckey(row_v[pl.ds(i * L, L)])

    cnt = jnp.where(cnt < K, N, cnt)
    nv = (cnt + (L - 1)) // L

    def bit_body(bi, p):
        bit = jnp.left_shift(jnp.int32(1), 31 - bi)
        cand_t = p | bit
        cs = cand_t ^ SIGN

        def cbody(i, acc):
            kv = cand_v[pl.ds(i * L, L)]
            return acc + jnp.where(kv >= cs,
                                   jnp.full((L,), 1, jnp.int32),
                                   jnp.full((L,), 0, jnp.int32))

        accv = lax.fori_loop(0, nv, cbody, jnp.zeros((L,), jnp.int32))
        return jnp.where(jnp.sum(accv) >= K, cand_t, p)

    p_u = lax.fori_loop(0, 32, bit_body, jnp.int32(0))
    return p_u ^ SIGN


def _sc_body(x_hbm, out_hbm, row0_v, row1_v, cand_v,
             in_sem0, in_sem1, out_sem0, out_sem1):
    wid = lax.axis_index("s") * NC + lax.axis_index("c")
    r0 = wid * ROWS_PER_W
    bufs = [row0_v, row1_v]
    in_sems = [in_sem0, in_sem1]
    out_sems = [out_sem0, out_sem1]

    copies_in = [None] * ROWS_PER_W
    copies_out = [None] * ROWS_PER_W
    copies_in[0] = pltpu.async_copy(x_hbm.at[r0], bufs[0], in_sems[0])
    for j in range(ROWS_PER_W):
        b = j % 2
        row_v = bufs[b]
        copies_in[j].wait()
        if j + 1 < ROWS_PER_W:
            if j >= 1:
                copies_out[j - 1].wait()
            copies_in[j + 1] = pltpu.async_copy(
                x_hbm.at[r0 + j + 1], bufs[1 - b], in_sems[1 - b])

        vkey = _row_threshold_key(row_v, cand_v)

        @plsc.parallel_loop(0, NV, unroll=UNROLL)
        def _mask(i):
            v = row_v[pl.ds(i * L, L)]
            row_v[pl.ds(i * L, L)] = jnp.where(_ckey(v) > vkey, v,
                                               jnp.float32(0.0))

        copies_out[j] = pltpu.async_copy(row_v, out_hbm.at[r0 + j],
                                         out_sems[b])
    copies_out[ROWS_PER_W - 2].wait()
    copies_out[ROWS_PER_W - 1].wait()


@jax.jit
def _ksparse_sc(x):
    mesh = plsc.VectorSubcoreMesh(core_axis_name="c", subcore_axis_name="s")
    return pl.kernel(
        _sc_body,
        out_type=jax.ShapeDtypeStruct((R, N), jnp.float32),
        mesh=mesh,
        compiler_params=pltpu.CompilerParams(needs_layout_passes=False),
        scratch_types=[
            pltpu.VMEM((N,), jnp.float32),
            pltpu.VMEM((N,), jnp.float32),
            pltpu.VMEM((N + L,), jnp.int32),
            pltpu.SemaphoreType.DMA,
            pltpu.SemaphoreType.DMA,
            pltpu.SemaphoreType.DMA,
            pltpu.SemaphoreType.DMA,
        ],
    )(x)


def kernel(inputs):
    return _ksparse_sc(inputs)


# speedup vs baseline: 36.9568x; 1.1163x over previous
import jax
import jax.numpy as jnp
import numpy as np
from jax import lax
from jax.experimental import pallas as pl
from jax.experimental.pallas import tpu as pltpu
from jax.experimental.pallas import tpu_sc as plsc

R = 128
N = 32768
K = 65
L = 16
NV = N // L
NC = 2
NS = 16
NW = NC * NS
ROWS_PER_W = R // NW
PIVOT = np.float32(2.5)
SIGN = np.int32(-(2**31))
LOW31 = np.int32(0x7FFFFFFF)
INT_MIN = np.int32(-(2**31))
UNROLL = 8


def _ckey(v):
    b = lax.bitcast_convert_type(v, jnp.int32)
    return jnp.where(b >= 0, b, b ^ LOW31)


def _row_threshold(row_v, cand_v):
    @plsc.parallel_loop(0, NV, unroll=UNROLL,
                        carry=jnp.zeros((L,), jnp.int32))
    def offv(i, off):
        v = row_v[pl.ds(i * L, L)]
        m = v > PIVOT
        mi = m.astype(jnp.int32)
        pos = (plsc.cumsum(mi) - mi) + off
        plsc.store_scatter(cand_v, [pos], v, mask=m)
        return off + plsc.all_reduce_population_count(m)

    cnt = jnp.max(offv)
    padpos = lax.iota(jnp.int32, L) + cnt
    plsc.store_scatter(cand_v, [padpos],
                       jnp.full((L,), -jnp.inf, jnp.float32))

    @pl.when(cnt < K)
    def _():
        @plsc.parallel_loop(0, NV, unroll=UNROLL)
        def _copy(i):
            cand_v[pl.ds(i * L, L)] = row_v[pl.ds(i * L, L)]

    cnt = jnp.where(cnt < K, N, cnt)
    nv = (cnt + (L - 1)) // L

    @plsc.parallel_loop(0, nv, unroll=4)
    def _tokey(i):
        kv = _ckey(cand_v[pl.ds(i * L, L)])
        cand_v[pl.ds(i * L, L)] = lax.bitcast_convert_type(kv, jnp.float32)

    def bit_body(bi, p):
        bit = jnp.left_shift(jnp.int32(1), 31 - bi)
        cand_t = p | bit
        cs = cand_t ^ SIGN

        def cbody(i, acc):
            kv = lax.bitcast_convert_type(cand_v[pl.ds(i * L, L)], jnp.int32)
            return acc + jnp.where(kv >= cs,
                                   jnp.full((L,), 1, jnp.int32),
                                   jnp.full((L,), 0, jnp.int32))

        accv = lax.fori_loop(0, nv, cbody, jnp.zeros((L,), jnp.int32))
        return jnp.where(jnp.sum(accv) >= K, cand_t, p)

    p_u = lax.fori_loop(0, 32, bit_body, jnp.int32(0))
    vkey = p_u ^ SIGN
    bsplat = jnp.full((L,), vkey, jnp.int32)
    bsplat = jnp.where(bsplat >= 0, bsplat, bsplat ^ LOW31)
    return lax.bitcast_convert_type(bsplat, jnp.float32)


def _sc_body(x_hbm, out_hbm, row0_v, row1_v, cand_v,
             in_sem0, in_sem1, out_sem0, out_sem1):
    wid = lax.axis_index("s") * NC + lax.axis_index("c")
    r0 = wid * ROWS_PER_W
    bufs = [row0_v, row1_v]
    in_sems = [in_sem0, in_sem1]
    out_sems = [out_sem0, out_sem1]

    copies_in = [None] * ROWS_PER_W
    copies_out = [None] * ROWS_PER_W
    copies_in[0] = pltpu.async_copy(x_hbm.at[r0], bufs[0], in_sems[0])
    for j in range(ROWS_PER_W):
        b = j % 2
        row_v = bufs[b]
        copies_in[j].wait()
        if j + 1 < ROWS_PER_W:
            if j >= 1:
                copies_out[j - 1].wait()
            copies_in[j + 1] = pltpu.async_copy(
                x_hbm.at[r0 + j + 1], bufs[1 - b], in_sems[1 - b])

        thr = _row_threshold(row_v, cand_v)

        @plsc.parallel_loop(0, NV, unroll=UNROLL)
        def _mask(i):
            v = row_v[pl.ds(i * L, L)]
            row_v[pl.ds(i * L, L)] = jnp.where(v > thr, v, jnp.float32(0.0))

        copies_out[j] = pltpu.async_copy(row_v, out_hbm.at[r0 + j],
                                         out_sems[b])
    copies_out[ROWS_PER_W - 2].wait()
    copies_out[ROWS_PER_W - 1].wait()


@jax.jit
def _ksparse_sc(x):
    mesh = plsc.VectorSubcoreMesh(core_axis_name="c", subcore_axis_name="s")
    return pl.kernel(
        _sc_body,
        out_type=jax.ShapeDtypeStruct((R, N), jnp.float32),
        mesh=mesh,
        compiler_params=pltpu.CompilerParams(needs_layout_passes=False),
        scratch_types=[
            pltpu.VMEM((N,), jnp.float32),
            pltpu.VMEM((N,), jnp.float32),
            pltpu.VMEM((N + L,), jnp.float32),
            pltpu.SemaphoreType.DMA,
            pltpu.SemaphoreType.DMA,
            pltpu.SemaphoreType.DMA,
            pltpu.SemaphoreType.DMA,
        ],
    )(x)


def kernel(inputs):
    return _ksparse_sc(inputs)
